# baseline (speedup 1.0000x reference)
import jax
import jax.numpy as jnp
from jax import lax
from jax.experimental import pallas as pl
from jax.experimental.pallas import tpu as pltpu
from jax.experimental.pallas import tpu_sc as plsc

FRAME = 80
PMAX = 300
BATCH = 128
NF = 1000
NS = FRAME * NF
LANES = 16
NFP = 1024
NWORKERS = 32
RPW = BATCH // NWORKERS

HALO = 384
F0 = (0, 256, 512, 768)
NT = (256, 256, 256, 232)
NG = (16, 16, 16, 15)
SRC = (0, 80 * 256 - HALO, 80 * 512 - HALO, 80 * 768 - HALO)
LEN = (80 * 256, 80 * 512 - SRC[1], 80 * 768 - SRC[2], NS - SRC[3])
DST = (HALO, 0, 0, 0)
XBUF = HALO + LEN[1]


def _rsqrt(v):
    i = plsc.bitcast(v, jnp.int32)
    i = 0x5F3759DF - lax.shift_right_arithmetic(i, 1)
    y = plsc.bitcast(i, jnp.float32)
    for _ in range(3):
        y = y * (1.5 - 0.5 * v * y * y)
    return y


def _sc_body(x_hbm, per_hbm, out_hbm, xb0, xb1, pv, out_v, dv, ev, gv,
             sem0, sem1):
    cid = lax.axis_index("c")
    sid = lax.axis_index("s")
    wid = sid * 2 + cid
    iota = lax.iota(jnp.int32, LANES)
    xbufs = (xb0, xb1)
    sems = (sem0, sem1)
    zero = jnp.zeros((LANES,), jnp.float32)
    m15 = iota == (LANES - 1)

    pltpu.sync_copy(per_hbm.at[pl.ds(wid * RPW * NFP, RPW * NFP)], pv)

    def copy_refs(q, b):
        buf = xbufs[q % 2]
        return (x_hbm.at[b, 0, pl.ds(SRC[q], LEN[q])],
                buf.at[pl.ds(DST[q], LEN[q])])

    def start(q, b):
        src, dst = copy_refs(q, b)
        pltpu.async_copy(src, dst, sems[q % 2])

    def wait(q, b):
        src, dst = copy_refs(q, b)
        pltpu.make_async_copy(src, dst, sems[q % 2]).wait()

    def compute(q, r, b):
        buf = xbufs[q % 2]
        if q == 0:
            for z in range(HALO // LANES):
                buf[pl.ds(z * LANES, LANES)] = zero

        pbase = r * NFP + F0[q]

        @plsc.parallel_loop(0, NT[q], 1, unroll=8)
        def frame_body(f):
            p = pv[pl.ds(pbase + f, LANES)][0]
            fbase = f * FRAME + HALO
            lbase = fbase - p
            fv0 = buf[pl.ds(fbase, LANES)]
            fv1 = buf[pl.ds(fbase + 16, LANES)]
            fv2 = buf[pl.ds(fbase + 32, LANES)]
            fv3 = buf[pl.ds(fbase + 48, LANES)]
            fv4 = buf[pl.ds(fbase + 64, LANES)]
            lv0 = buf[pl.ds(lbase, LANES)]
            lv1 = buf[pl.ds(lbase + 16, LANES)]
            lv2 = buf[pl.ds(lbase + 32, LANES)]
            lv3 = buf[pl.ds(lbase + 48, LANES)]
            lv4 = buf[pl.ds(lbase + 64, LANES)]
            da = (fv0 * lv0 + fv1 * lv1) + (fv2 * lv2 + fv3 * lv3) + fv4 * lv4
            ea = (fv0 * fv0 + fv1 * fv1) + (fv2 * fv2 + fv3 * fv3) + fv4 * fv4
            ga = (lv0 * lv0 + lv1 * lv1) + (lv2 * lv2 + lv3 * lv3) + lv4 * lv4
            plsc.store_compressed(dv.at[pl.ds(f, LANES)], plsc.cumsum(da), mask=m15)
            plsc.store_compressed(ev.at[pl.ds(f, LANES)], plsc.cumsum(ea), mask=m15)
            plsc.store_compressed(gv.at[pl.ds(f, LANES)], plsc.cumsum(ga), mask=m15)

        @plsc.parallel_loop(0, NG[q], 1, unroll=2)
        def norm_body(g):
            dd = dv[pl.ds(g * LANES, LANES)]
            ff = ev[pl.ds(g * LANES, LANES)]
            ll = gv[pl.ds(g * LANES, LANES)]
            res = dd * _rsqrt(ff * ll + 1e-9)
            out_v[pl.ds(pbase + g * LANES, LANES)] = res

    def row(r, carry):
        b = wid * RPW + r
        bnext = wid * RPW + jnp.minimum(r + 1, RPW - 1)
        start(1, b)
        wait(0, b)
        compute(0, r, b)
        start(2, b)
        wait(1, b)
        compute(1, r, b)
        start(3, b)
        wait(2, b)
        compute(2, r, b)
        start(0, bnext)
        wait(3, b)
        compute(3, r, b)
        return carry

    start(0, wid * RPW)
    lax.fori_loop(0, RPW, row, 0)
    wait(0, wid * RPW + RPW - 1)

    pltpu.sync_copy(out_v, out_hbm.at[pl.ds(wid * RPW * NFP, RPW * NFP)])


@jax.jit
def kernel(x, periods):
    pp = jnp.pad(periods, ((0, 0), (0, NFP - NF))).reshape(-1)
    run = pl.kernel(
        _sc_body,
        out_type=jax.ShapeDtypeStruct((BATCH * NFP,), jnp.float32),
        mesh=plsc.VectorSubcoreMesh(core_axis_name="c", subcore_axis_name="s"),
        scratch_types=[
            pltpu.VMEM((XBUF,), jnp.float32),
            pltpu.VMEM((XBUF,), jnp.float32),
            pltpu.VMEM((RPW * NFP,), jnp.int32),
            pltpu.VMEM((RPW * NFP,), jnp.float32),
            pltpu.VMEM((288,), jnp.float32),
            pltpu.VMEM((288,), jnp.float32),
            pltpu.VMEM((288,), jnp.float32),
            pltpu.SemaphoreType.DMA,
            pltpu.SemaphoreType.DMA,
        ],
        compiler_params=pltpu.CompilerParams(needs_layout_passes=False),
    )
    out = run(x, pp)
    return out.reshape(BATCH, NFP)[:, :NF].reshape(BATCH, 1, NF, 1)


# speedup vs baseline: 1.2013x; 1.2013x over previous
import jax
import jax.numpy as jnp
from jax import lax
from jax.experimental import pallas as pl
from jax.experimental.pallas import tpu as pltpu
from jax.experimental.pallas import tpu_sc as plsc

FRAME = 80
PMAX = 300
BATCH = 128
NF = 1000
NS = FRAME * NF
LANES = 16
NFP = 1024
NWORKERS = 32
RPW = BATCH // NWORKERS

HALO = 384
F0 = (0, 256, 512, 768)
NT = (256, 256, 256, 232)
NG = (16, 16, 16, 15)
SRC = (0, 80 * 256 - HALO, 80 * 512 - HALO, 80 * 768 - HALO)
LEN = (80 * 256, 80 * 512 - SRC[1], 80 * 768 - SRC[2], NS - SRC[3])
DST = (HALO, 0, 0, 0)
XBUF = HALO + LEN[1]


def _rsqrt(v):
    i = plsc.bitcast(v, jnp.int32)
    i = 0x5F3759DF - lax.shift_right_arithmetic(i, 1)
    y = plsc.bitcast(i, jnp.float32)
    for _ in range(3):
        y = y * (1.5 - 0.5 * v * y * y)
    return y


def _sc_body(x_hbm, per_hbm, out_hbm, xb0, xb1, pv, out_v, dv, ev, gv,
             sem0, sem1):
    cid = lax.axis_index("c")
    sid = lax.axis_index("s")
    wid = sid * 2 + cid
    iota = lax.iota(jnp.int32, LANES)
    xbufs = (xb0, xb1)
    sems = (sem0, sem1)
    zero = jnp.zeros((LANES,), jnp.float32)
    m15 = iota == (LANES - 1)

    pltpu.sync_copy(per_hbm.at[pl.ds(wid * RPW * NFP, RPW * NFP)], pv)

    def copy_refs(q, b):
        buf = xbufs[q % 2]
        return (x_hbm.at[b, 0, pl.ds(SRC[q], LEN[q])],
                buf.at[pl.ds(DST[q], LEN[q])])

    def start(q, b):
        src, dst = copy_refs(q, b)
        pltpu.async_copy(src, dst, sems[q % 2])

    def wait(q, b):
        src, dst = copy_refs(q, b)
        pltpu.make_async_copy(src, dst, sems[q % 2]).wait()

    def compute(q, r, b):
        buf = xbufs[q % 2]
        if q == 0:
            for z in range(HALO // LANES):
                buf[pl.ds(z * LANES, LANES)] = zero

        pbase = r * NFP + F0[q]

        @plsc.parallel_loop(0, NT[q], 1, unroll=2)
        def frame_body(f):
            p = pv[pl.ds(pbase + f, LANES)][0]
            fbase = f * FRAME + HALO
            lbase = fbase - p
            fv0 = buf[pl.ds(fbase, LANES)]
            fv1 = buf[pl.ds(fbase + 16, LANES)]
            fv2 = buf[pl.ds(fbase + 32, LANES)]
            fv3 = buf[pl.ds(fbase + 48, LANES)]
            fv4 = buf[pl.ds(fbase + 64, LANES)]
            lv0 = buf[pl.ds(lbase, LANES)]
            lv1 = buf[pl.ds(lbase + 16, LANES)]
            lv2 = buf[pl.ds(lbase + 32, LANES)]
            lv3 = buf[pl.ds(lbase + 48, LANES)]
            lv4 = buf[pl.ds(lbase + 64, LANES)]
            da = (fv0 * lv0 + fv1 * lv1) + (fv2 * lv2 + fv3 * lv3) + fv4 * lv4
            ea = (fv0 * fv0 + fv1 * fv1) + (fv2 * fv2 + fv3 * fv3) + fv4 * fv4
            ga = (lv0 * lv0 + lv1 * lv1) + (lv2 * lv2 + lv3 * lv3) + lv4 * lv4
            plsc.store_compressed(dv.at[pl.ds(f, LANES)], plsc.cumsum(da), mask=m15)
            plsc.store_compressed(ev.at[pl.ds(f, LANES)], plsc.cumsum(ea), mask=m15)
            plsc.store_compressed(gv.at[pl.ds(f, LANES)], plsc.cumsum(ga), mask=m15)

        @plsc.parallel_loop(0, NG[q], 1, unroll=2)
        def norm_body(g):
            dd = dv[pl.ds(g * LANES, LANES)]
            ff = ev[pl.ds(g * LANES, LANES)]
            ll = gv[pl.ds(g * LANES, LANES)]
            res = dd * _rsqrt(ff * ll + 1e-9)
            out_v[pl.ds(pbase + g * LANES, LANES)] = res

    def row(r, carry):
        b = wid * RPW + r
        bnext = wid * RPW + jnp.minimum(r + 1, RPW - 1)
        start(1, b)
        wait(0, b)
        compute(0, r, b)
        start(2, b)
        wait(1, b)
        compute(1, r, b)
        start(3, b)
        wait(2, b)
        compute(2, r, b)
        start(0, bnext)
        wait(3, b)
        compute(3, r, b)
        return carry

    start(0, wid * RPW)
    lax.fori_loop(0, RPW, row, 0)
    wait(0, wid * RPW + RPW - 1)

    pltpu.sync_copy(out_v, out_hbm.at[pl.ds(wid * RPW * NFP, RPW * NFP)])


@jax.jit
def kernel(x, periods):
    pp = jnp.pad(periods, ((0, 0), (0, NFP - NF))).reshape(-1)
    run = pl.kernel(
        _sc_body,
        out_type=jax.ShapeDtypeStruct((BATCH * NFP,), jnp.float32),
        mesh=plsc.VectorSubcoreMesh(core_axis_name="c", subcore_axis_name="s"),
        scratch_types=[
            pltpu.VMEM((XBUF,), jnp.float32),
            pltpu.VMEM((XBUF,), jnp.float32),
            pltpu.VMEM((RPW * NFP,), jnp.int32),
            pltpu.VMEM((RPW * NFP,), jnp.float32),
            pltpu.VMEM((288,), jnp.float32),
            pltpu.VMEM((288,), jnp.float32),
            pltpu.VMEM((288,), jnp.float32),
            pltpu.SemaphoreType.DMA,
            pltpu.SemaphoreType.DMA,
        ],
        compiler_params=pltpu.CompilerParams(needs_layout_passes=False),
    )
    out = run(x, pp)
    return out.reshape(BATCH, NFP)[:, :NF].reshape(BATCH, 1, NF, 1)


# speedup vs baseline: 1.2113x; 1.0083x over previous
import jax
import jax.numpy as jnp
from jax import lax
from jax.experimental import pallas as pl
from jax.experimental.pallas import tpu as pltpu
from jax.experimental.pallas import tpu_sc as plsc

FRAME = 80
PMAX = 300
BATCH = 128
NF = 1000
NS = FRAME * NF
LANES = 16
NFP = 1024
NWORKERS = 32
RPW = BATCH // NWORKERS

HALO = 384
F0 = (0, 256, 512, 768)
NT = (256, 256, 256, 232)
NG = (16, 16, 16, 15)
SRC = (0, 80 * 256 - HALO, 80 * 512 - HALO, 80 * 768 - HALO)
LEN = (80 * 256, 80 * 512 - SRC[1], 80 * 768 - SRC[2], NS - SRC[3])
DST = (HALO, 0, 0, 0)
XBUF = HALO + LEN[1]


def _rsqrt(v):
    i = plsc.bitcast(v, jnp.int32)
    i = 0x5F3759DF - lax.shift_right_arithmetic(i, 1)
    y = plsc.bitcast(i, jnp.float32)
    for _ in range(3):
        y = y * (1.5 - 0.5 * v * y * y)
    return y


def _sc_body(x_hbm, per_hbm, out_hbm, xb0, xb1, pv, out_v, dv, ev, gv,
             sem0, sem1):
    cid = lax.axis_index("c")
    sid = lax.axis_index("s")
    wid = sid * 2 + cid
    iota = lax.iota(jnp.int32, LANES)
    xbufs = (xb0, xb1)
    sems = (sem0, sem1)
    zero = jnp.zeros((LANES,), jnp.float32)
    m15 = iota == (LANES - 1)

    pltpu.sync_copy(per_hbm.at[pl.ds(wid * RPW * NFP, RPW * NFP)], pv)

    def copy_refs(q, b):
        buf = xbufs[q % 2]
        return (x_hbm.at[b, 0, pl.ds(SRC[q], LEN[q])],
                buf.at[pl.ds(DST[q], LEN[q])])

    def start(q, b):
        src, dst = copy_refs(q, b)
        pltpu.async_copy(src, dst, sems[q % 2])

    def wait(q, b):
        src, dst = copy_refs(q, b)
        pltpu.make_async_copy(src, dst, sems[q % 2]).wait()

    def compute(q, r, b):
        buf = xbufs[q % 2]
        if q == 0:
            for z in range(HALO // LANES):
                buf[pl.ds(z * LANES, LANES)] = zero

        pbase = r * NFP + F0[q]

        @plsc.parallel_loop(0, NT[q], 1, unroll=4)
        def frame_body(f):
            p = pv[pl.ds(pbase + f, LANES)][0]
            fbase = f * FRAME + HALO
            lbase = fbase - p
            fv0 = buf[pl.ds(fbase, LANES)]
            fv1 = buf[pl.ds(fbase + 16, LANES)]
            fv2 = buf[pl.ds(fbase + 32, LANES)]
            fv3 = buf[pl.ds(fbase + 48, LANES)]
            fv4 = buf[pl.ds(fbase + 64, LANES)]
            lv0 = buf[pl.ds(lbase, LANES)]
            lv1 = buf[pl.ds(lbase + 16, LANES)]
            lv2 = buf[pl.ds(lbase + 32, LANES)]
            lv3 = buf[pl.ds(lbase + 48, LANES)]
            lv4 = buf[pl.ds(lbase + 64, LANES)]
            da = (fv0 * lv0 + fv1 * lv1) + (fv2 * lv2 + fv3 * lv3) + fv4 * lv4
            ea = (fv0 * fv0 + fv1 * fv1) + (fv2 * fv2 + fv3 * fv3) + fv4 * fv4
            ga = (lv0 * lv0 + lv1 * lv1) + (lv2 * lv2 + lv3 * lv3) + lv4 * lv4
            plsc.store_compressed(dv.at[pl.ds(f, LANES)], plsc.cumsum(da), mask=m15)
            plsc.store_compressed(ev.at[pl.ds(f, LANES)], plsc.cumsum(ea), mask=m15)
            plsc.store_compressed(gv.at[pl.ds(f, LANES)], plsc.cumsum(ga), mask=m15)

        @plsc.parallel_loop(0, NG[q], 1, unroll=2)
        def norm_body(g):
            dd = dv[pl.ds(g * LANES, LANES)]
            ff = ev[pl.ds(g * LANES, LANES)]
            ll = gv[pl.ds(g * LANES, LANES)]
            res = dd * _rsqrt(ff * ll + 1e-9)
            out_v[pl.ds(pbase + g * LANES, LANES)] = res

    def row(r, carry):
        b = wid * RPW + r
        bnext = wid * RPW + jnp.minimum(r + 1, RPW - 1)
        start(1, b)
        wait(0, b)
        compute(0, r, b)
        start(2, b)
        wait(1, b)
        compute(1, r, b)
        start(3, b)
        wait(2, b)
        compute(2, r, b)
        start(0, bnext)
        wait(3, b)
        compute(3, r, b)
        return carry

    start(0, wid * RPW)
    lax.fori_loop(0, RPW, row, 0)
    wait(0, wid * RPW + RPW - 1)

    pltpu.sync_copy(out_v, out_hbm.at[pl.ds(wid * RPW * NFP, RPW * NFP)])


@jax.jit
def kernel(x, periods):
    pp = jnp.pad(periods, ((0, 0), (0, NFP - NF))).reshape(-1)
    run = pl.kernel(
        _sc_body,
        out_type=jax.ShapeDtypeStruct((BATCH * NFP,), jnp.float32),
        mesh=plsc.VectorSubcoreMesh(core_axis_name="c", subcore_axis_name="s"),
        scratch_types=[
            pltpu.VMEM((XBUF,), jnp.float32),
            pltpu.VMEM((XBUF,), jnp.float32),
            pltpu.VMEM((RPW * NFP,), jnp.int32),
            pltpu.VMEM((RPW * NFP,), jnp.float32),
            pltpu.VMEM((288,), jnp.float32),
            pltpu.VMEM((288,), jnp.float32),
            pltpu.VMEM((288,), jnp.float32),
            pltpu.SemaphoreType.DMA,
            pltpu.SemaphoreType.DMA,
        ],
        compiler_params=pltpu.CompilerParams(needs_layout_passes=False),
    )
    out = run(x, pp)
    return out.reshape(BATCH, NFP)[:, :NF].reshape(BATCH, 1, NF, 1)
